# baseline (speedup 1.0000x reference)
import functools

import jax
import jax.numpy as jnp
from jax import lax
from jax.experimental import pallas as pl
from jax.experimental.pallas import tpu as pltpu
from jax.experimental.pallas import tpu_sc as plsc

_B, _N = 4, 20000
_K = 128
_KIOU = 64
_NW = 32
_WCHUNK = 2560
_NSL = _WCHUNK // 16
_NSL_TAIL = (_N - 7 * _WCHUNK) // 16
_CPW = 64
_CAND = 8 * _CPW
_NROUNDS = 4
_H = _W = 16
_P = _H * _W

_MESH = plsc.VectorSubcoreMesh(core_axis_name="c", subcore_axis_name="s")
_CP_SC = pltpu.CompilerParams(use_tc_tiling_on_sc=False,
                              needs_layout_passes=False)


@functools.partial(
    pl.kernel, mesh=_MESH, compiler_params=_CP_SC,
    out_type=(jax.ShapeDtypeStruct((_NW * _CPW,), jnp.float32),
              jax.ShapeDtypeStruct((_NW * _CPW,), jnp.int32)),
    scratch_types=[pltpu.VMEM((_WCHUNK,), jnp.float32),
                   pltpu.VMEM((_CPW,), jnp.float32),
                   pltpu.VMEM((_CPW,), jnp.int32),
                   pltpu.VMEM((16,), jnp.int32),
                   pltpu.VMEM((8, 16), jnp.int32),
                   pltpu.VMEM_SHARED((_NROUNDS, 16, 16), jnp.int32)],
)
def _sc_topk(conf_hbm, oval_hbm, oidx_hbm, cval, cv, ci, cntv, tmp, shared):
    c = lax.axis_index("c")
    s = lax.axis_index("s")
    g = s // 8
    t = s % 8
    b = 2 * c + g
    row = 16 * c + s
    nsl = _NSL
    pltpu.sync_copy(conf_hbm.at[row], cval)

    li = lax.broadcasted_iota(jnp.int32, (16,), 0)
    lo = jnp.float32(0.0)
    width = jnp.float32(1.0)
    for r in range(_NROUNDS):
        q = width * jnp.float32(0.25)
        t1 = lo + q
        t2 = lo + q * jnp.float32(2.0)
        t3 = lo + q * jnp.float32(3.0)

        def body(i, carry):
            a1, a2, a3 = carry
            for u in range(4):
                x = cval[pl.ds(i * 64 + u * 16, 16)]
                a1 = a1 + (x >= t1).astype(jnp.int32)
                a2 = a2 + (x >= t2).astype(jnp.int32)
                a3 = a3 + (x >= t3).astype(jnp.int32)
            return a1, a2, a3

        z = jnp.zeros((16,), jnp.int32)
        a1, a2, a3 = lax.fori_loop(0, nsl // 4, body, (z, z, z))
        n1 = jnp.sum(a1)
        n2 = jnp.sum(a2)
        n3 = jnp.sum(a3)
        cntv[...] = jnp.where(li == 0, n1,
                              jnp.where(li == 1, n2,
                                        jnp.where(li == 2, n3, 0)))
        pltpu.sync_copy(cntv, shared.at[r, s])
        plsc.subcore_barrier()
        pltpu.sync_copy(shared.at[r, pl.ds(8 * g, 8)], tmp)
        gcnt = jnp.zeros((16,), jnp.int32)
        for rr in range(8):
            gcnt = gcnt + tmp[rr]
        sel = jnp.sum(((gcnt >= _K) & (li < 3)).astype(jnp.int32))
        lo = lo + q * sel.astype(jnp.float32)
        width = q

    for i in range(_CPW // 16):
        cv[pl.ds(16 * i, 16)] = jnp.full((16,), -1.0, jnp.float32)
        ci[pl.ds(16 * i, 16)] = jnp.zeros((16,), jnp.int32)
    base_idx = t * _WCHUNK
    lof = lo

    def cbody(i, off):
        x = cval[pl.ds(i * 16, 16)]
        m = x >= lof
        mi = m.astype(jnp.int32)
        pos = off + plsc.cumsum(mi) - mi
        m2 = m & (pos < _CPW)
        plsc.store_scatter(cv, [pos], x, mask=m2)
        gi = base_idx + i * 16 + li
        plsc.store_scatter(ci, [pos], gi, mask=m2)
        return off + jnp.sum(mi)

    lax.fori_loop(0, nsl, cbody, jnp.int32(0))

    obase = _CAND * b + _CPW * t
    pltpu.sync_copy(cv, oval_hbm.at[pl.ds(obase, _CPW)])
    pltpu.sync_copy(ci, oidx_hbm.at[pl.ds(obase, _CPW)])


def _sel_body(cv_ref, ci_ref, val_ref, idx_ref):
    cv = cv_ref[0, 0]
    ci = ci_ref[0, 0]
    gt = (cv[:, None] > cv[None, :]).astype(jnp.int32)
    tie = ((cv[:, None] == cv[None, :]) & (ci[:, None] < ci[None, :]))
    rank = jnp.sum(gt + tie.astype(jnp.int32), axis=0)
    r = lax.broadcasted_iota(jnp.int32, (_K, _CAND), 0)
    oh = rank[None, :] == r
    val_ref[0, 0] = jnp.sum(jnp.where(oh, cv[None, :], 0.0), axis=1)
    idx_ref[0, 0] = jnp.sum(jnp.where(oh, ci[None, :], 0), axis=1)


@functools.partial(
    pl.kernel, mesh=_MESH, compiler_params=_CP_SC,
    out_type=(jax.ShapeDtypeStruct((_B * _K, 32), jnp.float32),
              jax.ShapeDtypeStruct((_B * _K * 4,), jnp.float32)),
    scratch_types=[pltpu.VMEM((16,), jnp.int32),
                   pltpu.VMEM((16,), jnp.int32),
                   pltpu.VMEM((16, 32), jnp.float32),
                   pltpu.VMEM((16, 16), jnp.float32),
                   pltpu.VMEM((64,), jnp.float32),
                   pltpu.SemaphoreType.DMA,
                   pltpu.SemaphoreType.DMA],
)
def _sc_gather(mask_hbm, loc4_hbm, idx_hbm, omask_hbm, oloc_hbm,
               idx_v, row_v, rm, rl, lout, sm, sl):
    wid = 16 * lax.axis_index("c") + lax.axis_index("s")
    base = wid * 16
    pltpu.sync_copy(idx_hbm.at[pl.ds(base, 16)], idx_v)
    idx_v[...] = idx_v[...] + (wid // 8) * _N
    cm = pltpu.async_copy(mask_hbm.at[idx_v], rm, sm)
    row_v[...] = idx_v[...] // 4
    cl = pltpu.async_copy(loc4_hbm.at[row_v], rl, sl)
    cm.wait()
    pltpu.sync_copy(rm, omask_hbm.at[pl.ds(base, 16)])
    cl.wait()
    lane = lax.broadcasted_iota(jnp.int32, (16,), 0)
    coord = lane % 4
    for k in range(4):
        jvec = lane // 4 + 4 * k
        gidx = plsc.load_gather(idx_v, [jvec])
        colv = (gidx % 4) * 4 + coord
        lout[pl.ds(16 * k, 16)] = plsc.load_gather(rl, [jvec, colv])
    pltpu.sync_copy(lout, oloc_hbm.at[pl.ds(base * 4, 64)])


def _iou_body(loc_ref, xg_ref, yg_ref, iou_ref, kidx_ref):
    locb = loc_ref[0]
    cx = locb[:, 0:1]
    cy = locb[:, 1:2]
    sx = jnp.abs(locb[:, 2:3]) + 1e-4
    sy = jnp.abs(locb[:, 3:4]) + 1e-4
    xg = xg_ref[...]
    yg = yg_ref[...]
    g = jnp.exp(-0.5 * (((xg - cx) / sx) ** 2 + ((yg - cy) / sy) ** 2))

    rows = []
    for i8 in range(_K // 8):
        gi = g[i8 * 8:(i8 + 1) * 8][:, None, :]
        inter = jnp.sum(jnp.minimum(gi, g[None]), axis=-1)
        union = jnp.sum(jnp.maximum(gi, g[None]), axis=-1)
        rows.append(inter / (union + 1e-9))
    iou = jnp.concatenate(rows, axis=0)
    iou_ref[0] = iou

    ii = lax.broadcasted_iota(jnp.int32, (_K, _K), 0)
    jj = lax.broadcasted_iota(jnp.int32, (_K, _K), 1)
    vmax = jnp.max(jnp.where(ii < jj, iou, 0.0), axis=0)

    lt = (vmax[:, None] < vmax[None, :]).astype(jnp.int32)
    tie = ((vmax[:, None] == vmax[None, :]) & (ii < jj)).astype(jnp.int32)
    rank = jnp.sum(lt + tie, axis=0)
    r = lax.broadcasted_iota(jnp.int32, (_KIOU, _K), 0)
    jcol = lax.broadcasted_iota(jnp.int32, (_KIOU, _K), 1)
    oh = rank[None, :] == r
    kidx_ref[0, 0] = jnp.sum(jnp.where(oh, jcol, 0), axis=1)


def kernel(conf, loc, mask):
    conf_p = jnp.pad(conf, ((0, 0), (0, 8 * _WCHUNK - _N)),
                     constant_values=-1.0).reshape(_NW, _WCHUNK)
    cval_flat, cidx_flat = _sc_topk(conf_p)

    sorted_conf3, sorted_idx3 = pl.pallas_call(
        _sel_body,
        grid=(_B,),
        in_specs=[pl.BlockSpec((1, 1, _CAND), lambda b: (b, 0, 0)),
                  pl.BlockSpec((1, 1, _CAND), lambda b: (b, 0, 0))],
        out_specs=[pl.BlockSpec((1, 1, _K), lambda b: (b, 0, 0)),
                   pl.BlockSpec((1, 1, _K), lambda b: (b, 0, 0))],
        out_shape=[jax.ShapeDtypeStruct((_B, 1, _K), jnp.float32),
                   jax.ShapeDtypeStruct((_B, 1, _K), jnp.int32)],
    )(cval_flat.reshape(_B, 1, _CAND), cidx_flat.reshape(_B, 1, _CAND))

    gidx = sorted_idx3.reshape(_B * _K)
    mask_flat = mask.reshape(_B * _N, 32)
    loc4 = loc.reshape(_B * _N // 4, 16)
    smask, sloc_flat = _sc_gather(mask_flat, loc4, gidx)
    sorted_mask = smask.reshape(_B, _K, 32)
    sorted_loc = sloc_flat.reshape(_B, _K, 4)

    ys = jnp.linspace(0.0, 1.0, _H, dtype=jnp.float32)
    xs = jnp.linspace(0.0, 1.0, _W, dtype=jnp.float32)
    yy, xx = jnp.meshgrid(ys, xs, indexing='ij')
    xg = xx.reshape(1, _P)
    yg = yy.reshape(1, _P)

    gauss_iou, sorted_iou_idx = pl.pallas_call(
        _iou_body,
        grid=(_B,),
        in_specs=[pl.BlockSpec((1, _K, 4), lambda b: (b, 0, 0)),
                  pl.BlockSpec((1, _P), lambda b: (0, 0)),
                  pl.BlockSpec((1, _P), lambda b: (0, 0))],
        out_specs=[pl.BlockSpec((1, _K, _K), lambda b: (b, 0, 0)),
                   pl.BlockSpec((1, 1, _KIOU), lambda b: (b, 0, 0))],
        out_shape=[jax.ShapeDtypeStruct((_B, _K, _K), jnp.float32),
                   jax.ShapeDtypeStruct((_B, 1, _KIOU), jnp.int32)],
    )(sorted_loc, xg, yg)

    return (gauss_iou, sorted_loc, sorted_mask,
            sorted_conf3.reshape(_B, _K), sorted_iou_idx.reshape(_B, _KIOU))


# speedup vs baseline: 1.0029x; 1.0029x over previous
import functools

import jax
import jax.numpy as jnp
from jax import lax
from jax.experimental import pallas as pl
from jax.experimental.pallas import tpu as pltpu
from jax.experimental.pallas import tpu_sc as plsc

_B, _N = 4, 20000
_K = 128
_KIOU = 64
_NW = 32
_WCHUNK = 2560
_NSL = _WCHUNK // 16
_CPW = 64
_CAND = 8 * _CPW
_NROUNDS = 4
_H = _W = 16
_P = _H * _W

_MESH = plsc.VectorSubcoreMesh(core_axis_name="c", subcore_axis_name="s")
_CP_SC = pltpu.CompilerParams(use_tc_tiling_on_sc=False,
                              needs_layout_passes=False)


@functools.partial(
    pl.kernel, mesh=_MESH, compiler_params=_CP_SC,
    out_type=(jax.ShapeDtypeStruct((_NW * _CPW,), jnp.float32),
              jax.ShapeDtypeStruct((_NW * _CPW,), jnp.int32)),
    scratch_types=[pltpu.VMEM((_WCHUNK,), jnp.float32),
                   pltpu.VMEM((_CPW,), jnp.float32),
                   pltpu.VMEM((_CPW,), jnp.int32),
                   pltpu.VMEM((16,), jnp.int32),
                   pltpu.VMEM((8, 16), jnp.int32),
                   pltpu.VMEM_SHARED((_NROUNDS, 16, 16), jnp.int32)],
)
def _sc_topk(conf_hbm, oval_hbm, oidx_hbm, cval, cv, ci, cntv, tmp, shared):
    c = lax.axis_index("c")
    s = lax.axis_index("s")
    g = s // 8
    t = s % 8
    b = 2 * c + g
    row = 16 * c + s
    pltpu.sync_copy(conf_hbm.at[row], cval)

    li = lax.broadcasted_iota(jnp.int32, (16,), 0)
    lo = jnp.float32(0.0)
    width = jnp.float32(1.0)
    for r in range(_NROUNDS):
        q = width * jnp.float32(0.25)
        t1 = lo + q
        t2 = lo + q * jnp.float32(2.0)
        t3 = lo + q * jnp.float32(3.0)

        def body(i, carry):
            a1, a2, a3 = carry
            for u in range(4):
                x = cval[pl.ds(i * 64 + u * 16, 16)]
                a1 = a1 + (x >= t1).astype(jnp.int32)
                a2 = a2 + (x >= t2).astype(jnp.int32)
                a3 = a3 + (x >= t3).astype(jnp.int32)
            return a1, a2, a3

        z = jnp.zeros((16,), jnp.int32)
        a1, a2, a3 = lax.fori_loop(0, _NSL // 4, body, (z, z, z))
        n1 = jnp.sum(a1)
        n2 = jnp.sum(a2)
        n3 = jnp.sum(a3)
        cntv[...] = jnp.where(li == 0, n1,
                              jnp.where(li == 1, n2,
                                        jnp.where(li == 2, n3, 0)))
        pltpu.sync_copy(cntv, shared.at[r, s])
        plsc.subcore_barrier()
        pltpu.sync_copy(shared.at[r, pl.ds(8 * g, 8)], tmp)
        gcnt = jnp.zeros((16,), jnp.int32)
        for rr in range(8):
            gcnt = gcnt + tmp[rr]
        sel = jnp.sum(((gcnt >= _K) & (li < 3)).astype(jnp.int32))
        lo = lo + q * sel.astype(jnp.float32)
        width = q

    for i in range(_CPW // 16):
        cv[pl.ds(16 * i, 16)] = jnp.full((16,), -1.0, jnp.float32)
        ci[pl.ds(16 * i, 16)] = jnp.zeros((16,), jnp.int32)
    base_idx = t * _WCHUNK
    lof = lo

    def cbody(i, off):
        x = cval[pl.ds(i * 16, 16)]
        m = x >= lof
        mi = m.astype(jnp.int32)
        pos = off + plsc.cumsum(mi) - mi
        m2 = m & (pos < _CPW)
        plsc.store_scatter(cv, [pos], x, mask=m2)
        gi = base_idx + i * 16 + li
        plsc.store_scatter(ci, [pos], gi, mask=m2)
        return off + jnp.sum(mi)

    lax.fori_loop(0, _NSL, cbody, jnp.int32(0))

    obase = _CAND * b + _CPW * t
    pltpu.sync_copy(cv, oval_hbm.at[pl.ds(obase, _CPW)])
    pltpu.sync_copy(ci, oidx_hbm.at[pl.ds(obase, _CPW)])


def _sel_body(cv_ref, ci_ref, val_ref, idx_ref):
    cv = cv_ref[0, 0]
    ci = ci_ref[0, 0]
    gt = (cv[:, None] > cv[None, :]).astype(jnp.int32)
    tie = ((cv[:, None] == cv[None, :]) & (ci[:, None] < ci[None, :]))
    rank = jnp.sum(gt + tie.astype(jnp.int32), axis=0)
    r = lax.broadcasted_iota(jnp.int32, (_K, _CAND), 0)
    oh = rank[None, :] == r
    val_ref[0, 0] = jnp.sum(jnp.where(oh, cv[None, :], 0.0), axis=1)
    idx_ref[0, 0] = jnp.sum(jnp.where(oh, ci[None, :], 0), axis=1)


@functools.partial(
    pl.kernel, mesh=_MESH, compiler_params=_CP_SC,
    out_type=(jax.ShapeDtypeStruct((_B * _K, 32), jnp.float32),
              jax.ShapeDtypeStruct((_B * _K * 4,), jnp.float32)),
    scratch_types=[pltpu.VMEM((16,), jnp.int32),
                   pltpu.VMEM((16,), jnp.int32),
                   pltpu.VMEM((16, 32), jnp.float32),
                   pltpu.VMEM((16, 16), jnp.float32),
                   pltpu.VMEM((64,), jnp.float32),
                   pltpu.SemaphoreType.DMA,
                   pltpu.SemaphoreType.DMA],
)
def _sc_gather(mask_hbm, loc4_hbm, idx_hbm, omask_hbm, oloc_hbm,
               idx_v, row_v, rm, rl, lout, sm, sl):
    wid = 16 * lax.axis_index("c") + lax.axis_index("s")
    base = wid * 16
    pltpu.sync_copy(idx_hbm.at[pl.ds(base, 16)], idx_v)
    idx_v[...] = idx_v[...] + (wid // 8) * _N
    cm = pltpu.async_copy(mask_hbm.at[idx_v], rm, sm)
    row_v[...] = idx_v[...] // 4
    cl = pltpu.async_copy(loc4_hbm.at[row_v], rl, sl)
    cm.wait()
    pltpu.sync_copy(rm, omask_hbm.at[pl.ds(base, 16)])
    cl.wait()
    lane = lax.broadcasted_iota(jnp.int32, (16,), 0)
    coord = lane % 4
    for k in range(4):
        jvec = lane // 4 + 4 * k
        gidx = plsc.load_gather(idx_v, [jvec])
        colv = (gidx % 4) * 4 + coord
        lout[pl.ds(16 * k, 16)] = plsc.load_gather(rl, [jvec, colv])
    pltpu.sync_copy(lout, oloc_hbm.at[pl.ds(base * 4, 64)])


def _iou_body(loc_ref, xg_ref, yg_ref, iou_ref, kidx_ref):
    locb = loc_ref[0]
    cx = locb[:, 0:1]
    cy = locb[:, 1:2]
    sx = jnp.abs(locb[:, 2:3]) + 1e-4
    sy = jnp.abs(locb[:, 3:4]) + 1e-4
    xg = xg_ref[...]
    yg = yg_ref[...]
    g = jnp.exp(-0.5 * (((xg - cx) / sx) ** 2 + ((yg - cy) / sy) ** 2))

    rows = []
    for i8 in range(_K // 8):
        gi = g[i8 * 8:(i8 + 1) * 8][:, None, :]
        inter = jnp.sum(jnp.minimum(gi, g[None]), axis=-1)
        union = jnp.sum(jnp.maximum(gi, g[None]), axis=-1)
        rows.append(inter / (union + 1e-9))
    iou = jnp.concatenate(rows, axis=0)
    iou_ref[0] = iou

    ii = lax.broadcasted_iota(jnp.int32, (_K, _K), 0)
    jj = lax.broadcasted_iota(jnp.int32, (_K, _K), 1)
    vmax = jnp.max(jnp.where(ii < jj, iou, 0.0), axis=0)

    lt = (vmax[:, None] < vmax[None, :]).astype(jnp.int32)
    tie = ((vmax[:, None] == vmax[None, :]) & (ii < jj)).astype(jnp.int32)
    rank = jnp.sum(lt + tie, axis=0)
    r = lax.broadcasted_iota(jnp.int32, (_KIOU, _K), 0)
    jcol = lax.broadcasted_iota(jnp.int32, (_KIOU, _K), 1)
    oh = rank[None, :] == r
    kidx_ref[0, 0] = jnp.sum(jnp.where(oh, jcol, 0), axis=1)


def kernel(conf, loc, mask):
    conf_p = jnp.pad(conf, ((0, 0), (0, 8 * _WCHUNK - _N)),
                     constant_values=-1.0).reshape(_NW, _WCHUNK)
    cval_flat, cidx_flat = _sc_topk(conf_p)

    sorted_conf3, sorted_idx3 = pl.pallas_call(
        _sel_body,
        grid=(_B,),
        in_specs=[pl.BlockSpec((1, 1, _CAND), lambda b: (b, 0, 0)),
                  pl.BlockSpec((1, 1, _CAND), lambda b: (b, 0, 0))],
        out_specs=[pl.BlockSpec((1, 1, _K), lambda b: (b, 0, 0)),
                   pl.BlockSpec((1, 1, _K), lambda b: (b, 0, 0))],
        out_shape=[jax.ShapeDtypeStruct((_B, 1, _K), jnp.float32),
                   jax.ShapeDtypeStruct((_B, 1, _K), jnp.int32)],
    )(cval_flat.reshape(_B, 1, _CAND), cidx_flat.reshape(_B, 1, _CAND))

    gidx = sorted_idx3.reshape(_B * _K)
    mask_flat = mask.reshape(_B * _N, 32)
    loc4 = loc.reshape(_B * _N // 4, 16)
    smask, sloc_flat = _sc_gather(mask_flat, loc4, gidx)
    sorted_mask = smask.reshape(_B, _K, 32)
    sorted_loc = sloc_flat.reshape(_B, _K, 4)

    ys = jnp.linspace(0.0, 1.0, _H, dtype=jnp.float32)
    xs = jnp.linspace(0.0, 1.0, _W, dtype=jnp.float32)
    yy, xx = jnp.meshgrid(ys, xs, indexing='ij')
    xg = xx.reshape(1, _P)
    yg = yy.reshape(1, _P)

    gauss_iou, sorted_iou_idx = pl.pallas_call(
        _iou_body,
        grid=(_B,),
        in_specs=[pl.BlockSpec((1, _K, 4), lambda b: (b, 0, 0)),
                  pl.BlockSpec((1, _P), lambda b: (0, 0)),
                  pl.BlockSpec((1, _P), lambda b: (0, 0))],
        out_specs=[pl.BlockSpec((1, _K, _K), lambda b: (b, 0, 0)),
                   pl.BlockSpec((1, 1, _KIOU), lambda b: (b, 0, 0))],
        out_shape=[jax.ShapeDtypeStruct((_B, _K, _K), jnp.float32),
                   jax.ShapeDtypeStruct((_B, 1, _KIOU), jnp.int32)],
    )(sorted_loc, xg, yg)

    return (gauss_iou, sorted_loc, sorted_mask,
            sorted_conf3.reshape(_B, _K), sorted_iou_idx.reshape(_B, _KIOU))
